# baseline (speedup 1.0000x reference)
import functools

import jax
import jax.numpy as jnp
from jax import lax
from jax.experimental import pallas as pl
from jax.experimental.pallas import tpu as pltpu
from jax.experimental.pallas import tpu_sc as plsc

F = 26
V = 100000
E = 16
ND = 11
B = 16384
H1, H2, OUT = 128, 64, 10

NC, NS = 2, 16
NW = NC * NS
IPW = B // NW
IB = 128
NBLK = IPW // IB
PHASES = (6, 5, 5, 5, 5)
MAXPH = max(PHASES)


def _sc_body(table, idxs, lanes, bi_out, idx_v, lane_v, rows_v, sum_v, ssq_v,
             sem):
    wid = lax.axis_index("s") * NC + lax.axis_index("c")
    lane_iota = lax.iota(jnp.int32, 16)

    def block(blk, carry):
        blkg = wid * NBLK + blk
        f0 = 0
        for p, nf in enumerate(PHASES):
            n = nf * IB
            off = (blkg * F + f0) * IB
            pltpu.sync_copy(idxs.at[pl.ds(off, n)], idx_v.at[pl.ds(0, n)])
            pltpu.sync_copy(lanes.at[pl.ds(off, n)], lane_v.at[pl.ds(0, n)])
            descs = []
            for j in range(nf):
                descs.append(pltpu.async_copy(
                    table.at[idx_v.at[pl.ds(j * IB, IB)]],
                    rows_v.at[pl.ds(j * IB, IB), :], sem))
            for d in descs:
                d.wait()

            first, last = p == 0, p == len(PHASES) - 1

            def group(gi, c):
                base = gi * 16
                rws = [f * IB + base + lane_iota for f in range(nf)]
                cls = [lane_v[pl.ds(f * IB + base, 16)] for f in range(nf)]
                for e in range(E):
                    v = plsc.load_gather(rows_v, [rws[0], cls[0] + e])
                    s = v
                    q = v * v
                    for f in range(1, nf):
                        v = plsc.load_gather(rows_v, [rws[f], cls[f] + e])
                        s = s + v
                        q = q + v * v
                    if first:
                        sum_v[e, pl.ds(base, 16)] = s
                        ssq_v[e, pl.ds(base, 16)] = q
                    elif last:
                        st = sum_v[e, pl.ds(base, 16)] + s
                        qt = ssq_v[e, pl.ds(base, 16)] + q
                        sum_v[e, pl.ds(base, 16)] = 0.5 * (st * st - qt)
                    else:
                        sum_v[e, pl.ds(base, 16)] += s
                        ssq_v[e, pl.ds(base, 16)] += q
                return c

            lax.fori_loop(0, IB // 16, group, 0)
            f0 += nf
        pltpu.sync_copy(sum_v, bi_out.at[:, pl.ds(blkg * IB, IB)])
        return carry

    lax.fori_loop(0, NBLK, block, 0)


_sc_pool = functools.partial(
    pl.kernel,
    out_type=jax.ShapeDtypeStruct((E, B), jnp.float32),
    mesh=plsc.VectorSubcoreMesh(core_axis_name="c", subcore_axis_name="s"),
    scratch_types=[
        pltpu.VMEM((MAXPH * IB,), jnp.int32),
        pltpu.VMEM((MAXPH * IB,), jnp.int32),
        pltpu.VMEM((MAXPH * IB, 128), jnp.float32),
        pltpu.VMEM((E, IB), jnp.float32),
        pltpu.VMEM((E, IB), jnp.float32),
        pltpu.SemaphoreType.DMA,
    ],
    compiler_params=pltpu.CompilerParams(needs_layout_passes=False),
)(_sc_body)


CBV = 8192
NV = -(-V // CBV)


def _repack_body(in_ref, out_ref):
    out_ref[...] = jnp.transpose(in_ref[...], (0, 2, 1))


def _repack(tables):
    t = jnp.transpose(tables, (0, 2, 1))
    out = pl.pallas_call(
        _repack_body,
        grid=(F, NV),
        in_specs=[pl.BlockSpec((1, E, CBV), lambda f, i: (f, 0, i))],
        out_specs=pl.BlockSpec((1, CBV, E), lambda f, i: (f, i, 0)),
        out_shape=jax.ShapeDtypeStruct((F, V, E), jnp.float32),
    )(t)
    return out.reshape(F * V // 8, 128)


BM = 2048


def _mlp_body(dense_ref, bit_ref, w1a_ref, w1b_ref, b1_ref, w2_ref, b2_ref,
              w3_ref, b3_ref, out_ref):
    h = jnp.dot(dense_ref[...], w1a_ref[...], preferred_element_type=jnp.float32)
    h += lax.dot_general(bit_ref[...], w1b_ref[...],
                         (((0,), (0,)), ((), ())),
                         preferred_element_type=jnp.float32)
    h = jnp.maximum(h + b1_ref[...], 0.0)
    h = jnp.dot(h, w2_ref[...], preferred_element_type=jnp.float32)
    h = jnp.maximum(h + b2_ref[...], 0.0)
    out_ref[...] = (
        jnp.dot(h, w3_ref[...], preferred_element_type=jnp.float32)
        + b3_ref[...])


def _mlp(dense, bi_t, W1a, W1b, b1, W2, b2, W3, b3):
    grid = (B // BM,)
    full = lambda shape: pl.BlockSpec(shape, lambda i: (0, 0))
    return pl.pallas_call(
        _mlp_body,
        grid=grid,
        in_specs=[
            pl.BlockSpec((BM, ND), lambda i: (i, 0)),
            pl.BlockSpec((E, BM), lambda i: (0, i)),
            full((ND, H1)),
            full((E, H1)),
            full((1, H1)),
            full((H1, H2)),
            full((1, H2)),
            full((H2, OUT)),
            full((1, OUT)),
        ],
        out_specs=pl.BlockSpec((BM, OUT), lambda i: (i, 0)),
        out_shape=jax.ShapeDtypeStruct((B, OUT), jnp.float32),
    )(dense, bi_t, W1a, W1b, b1, W2, b2, W3, b3)


def _block_major(a):
    return a.reshape(B // IB, IB, F).transpose(0, 2, 1).reshape(-1)


def kernel(target_x, tables, W1, b1, W2, b2, W3, b3):
    dense = target_x[:, :ND]
    sparse = target_x[:, ND:].astype(jnp.int32)
    flat_idx = sparse + (jnp.arange(F, dtype=jnp.int32) * V)[None, :]
    idx_blocks = _block_major(flat_idx >> 3)
    lane_blocks = _block_major((flat_idx & 7) << 4)
    table_g = _repack(tables)

    bi_t = _sc_pool(table_g, idx_blocks, lane_blocks)

    return _mlp(dense, bi_t, W1[:ND], W1[ND:], b1[None, :], W2, b2[None, :],
                W3, b3[None, :])


# speedup vs baseline: 1.4943x; 1.4943x over previous
import functools

import jax
import jax.numpy as jnp
from jax import lax
from jax.experimental import pallas as pl
from jax.experimental.pallas import tpu as pltpu
from jax.experimental.pallas import tpu_sc as plsc

F = 26
V = 100000
E = 16
ND = 11
B = 16384
H1, H2, OUT = 128, 64, 10

NC, NS = 2, 16
NW = NC * NS
IPW = B // NW
IB = 128
NBLK = IPW // IB
PHASES = (6, 5, 5, 5, 5)
MAXPH = max(PHASES)


def _sc_body(table, idxs, lanes, bi_out, idx_v, lane_v, rows_v, sum_v, ssq_v,
             sem):
    wid = lax.axis_index("s") * NC + lax.axis_index("c")
    lane_iota = lax.iota(jnp.int32, 16)

    def block(blk, carry):
        blkg = wid * NBLK + blk
        f0 = 0
        for p, nf in enumerate(PHASES):
            n = nf * IB
            off = (blkg * F + f0) * IB
            pltpu.sync_copy(idxs.at[pl.ds(off, n)], idx_v.at[pl.ds(0, n)])
            pltpu.sync_copy(lanes.at[pl.ds(off, n)], lane_v.at[pl.ds(0, n)])
            descs = []
            for j in range(nf):
                descs.append(pltpu.async_copy(
                    table.at[idx_v.at[pl.ds(j * IB, IB)]],
                    rows_v.at[pl.ds(j * IB, IB), :], sem))
            for d in descs:
                d.wait()

            first, last = p == 0, p == len(PHASES) - 1

            def group(gi, c):
                base = gi * 16
                rws = [f * IB + base + lane_iota for f in range(nf)]
                cls = [lane_v[pl.ds(f * IB + base, 16)] for f in range(nf)]
                for e in range(E):
                    v = plsc.load_gather(rows_v, [rws[0], cls[0] + e])
                    s = v
                    q = v * v
                    for f in range(1, nf):
                        v = plsc.load_gather(rows_v, [rws[f], cls[f] + e])
                        s = s + v
                        q = q + v * v
                    if first:
                        sum_v[e, pl.ds(base, 16)] = s
                        ssq_v[e, pl.ds(base, 16)] = q
                    elif last:
                        st = sum_v[e, pl.ds(base, 16)] + s
                        qt = ssq_v[e, pl.ds(base, 16)] + q
                        sum_v[e, pl.ds(base, 16)] = 0.5 * (st * st - qt)
                    else:
                        sum_v[e, pl.ds(base, 16)] += s
                        ssq_v[e, pl.ds(base, 16)] += q
                return c

            lax.fori_loop(0, IB // 16, group, 0)
            f0 += nf
        pltpu.sync_copy(sum_v, bi_out.at[:, pl.ds(blkg * IB, IB)])
        return carry

    lax.fori_loop(0, NBLK, block, 0)


_sc_pool = functools.partial(
    pl.kernel,
    out_type=jax.ShapeDtypeStruct((E, B), jnp.float32),
    mesh=plsc.VectorSubcoreMesh(core_axis_name="c", subcore_axis_name="s"),
    scratch_types=[
        pltpu.VMEM((MAXPH * IB,), jnp.int32),
        pltpu.VMEM((MAXPH * IB,), jnp.int32),
        pltpu.VMEM((MAXPH * IB, 128), jnp.float32),
        pltpu.VMEM((E, IB), jnp.float32),
        pltpu.VMEM((E, IB), jnp.float32),
        pltpu.SemaphoreType.DMA,
    ],
    compiler_params=pltpu.CompilerParams(needs_layout_passes=False),
)(_sc_body)


CBV = 8192
NV = -(-V // CBV)
RPF = 12504


def _repack_body(in_ref, out_ref):
    a = in_ref[0]
    ii = lax.broadcasted_iota(jnp.int32, (E, E), 0)
    jj = lax.broadcasted_iota(jnp.int32, (E, E), 1)
    eye = (ii == jj).astype(jnp.float32)
    t = lax.dot_general(a, eye, (((0,), (0,)), ((), ())),
                        preferred_element_type=jnp.float32)
    t3 = t.reshape(CBV // 8, 8, E)
    out_ref[0] = jnp.concatenate([t3[:, j, :] for j in range(8)], axis=-1)


def _repack(tables):
    t = jnp.transpose(tables, (0, 2, 1))
    out = pl.pallas_call(
        _repack_body,
        grid=(F, NV),
        in_specs=[pl.BlockSpec((1, E, CBV), lambda f, i: (f, 0, i))],
        out_specs=pl.BlockSpec((1, CBV // 8, 128), lambda f, i: (f, i, 0)),
        out_shape=jax.ShapeDtypeStruct((F, RPF, 128), jnp.float32),
    )(t)
    return out.reshape(F * RPF, 128)


BM = 2048


def _mlp_body(dense_ref, bit_ref, w1a_ref, w1b_ref, b1_ref, w2_ref, b2_ref,
              w3_ref, b3_ref, out_ref):
    h = jnp.dot(dense_ref[...], w1a_ref[...], preferred_element_type=jnp.float32)
    h += lax.dot_general(bit_ref[...], w1b_ref[...],
                         (((0,), (0,)), ((), ())),
                         preferred_element_type=jnp.float32)
    h = jnp.maximum(h + b1_ref[...], 0.0)
    h = jnp.dot(h, w2_ref[...], preferred_element_type=jnp.float32)
    h = jnp.maximum(h + b2_ref[...], 0.0)
    out_ref[...] = (
        jnp.dot(h, w3_ref[...], preferred_element_type=jnp.float32)
        + b3_ref[...])


def _mlp(dense, bi_t, W1a, W1b, b1, W2, b2, W3, b3):
    grid = (B // BM,)
    full = lambda shape: pl.BlockSpec(shape, lambda i: (0, 0))
    return pl.pallas_call(
        _mlp_body,
        grid=grid,
        in_specs=[
            pl.BlockSpec((BM, ND), lambda i: (i, 0)),
            pl.BlockSpec((E, BM), lambda i: (0, i)),
            full((ND, H1)),
            full((E, H1)),
            full((1, H1)),
            full((H1, H2)),
            full((1, H2)),
            full((H2, OUT)),
            full((1, OUT)),
        ],
        out_specs=pl.BlockSpec((BM, OUT), lambda i: (i, 0)),
        out_shape=jax.ShapeDtypeStruct((B, OUT), jnp.float32),
    )(dense, bi_t, W1a, W1b, b1, W2, b2, W3, b3)


def _block_major(a):
    return a.reshape(B // IB, IB, F).transpose(0, 2, 1).reshape(-1)


def kernel(target_x, tables, W1, b1, W2, b2, W3, b3):
    dense = target_x[:, :ND]
    sparse = target_x[:, ND:].astype(jnp.int32)
    row_idx = (sparse >> 3) + (jnp.arange(F, dtype=jnp.int32) * RPF)[None, :]
    idx_blocks = _block_major(row_idx)
    lane_blocks = _block_major((sparse & 7) << 4)
    table_g = _repack(tables)

    bi_t = _sc_pool(table_g, idx_blocks, lane_blocks)

    return _mlp(dense, bi_t, W1[:ND], W1[ND:], b1[None, :], W2, b2[None, :],
                W3, b3[None, :])


# speedup vs baseline: 5.5785x; 3.7331x over previous
import functools

import jax
import jax.numpy as jnp
from jax import lax
from jax.experimental import pallas as pl
from jax.experimental.pallas import tpu as pltpu
from jax.experimental.pallas import tpu_sc as plsc

F = 26
V = 100000
E = 16
ND = 11
B = 16384
H1, H2, OUT = 128, 64, 10

NC, NS = 2, 16
HB = B // NC


def _sc_body(tt, codes, bi_out, row_v, codes_v, acc_s, acc_q, sem):
    e = lax.axis_index("s")
    ch = lax.axis_index("c")

    for f in range(F):
        pltpu.sync_copy(tt.at[f, pl.ds(e, 1), :], row_v)
        pltpu.sync_copy(codes.at[f, pl.ds(ch * HB, HB)], codes_v)

        if f == 0:
            def grp0(g, cr):
                sl = pl.ds(g * 16, 16)
                v = plsc.load_gather(row_v, [jnp.zeros((16,), jnp.int32),
                                             codes_v[sl]])
                acc_s[0, sl] = v
                acc_q[sl] = v * v
                return cr
            lax.fori_loop(0, HB // 16, grp0, 0)
        else:
            def grp(g, cr):
                sl = pl.ds(g * 16, 16)
                v = plsc.load_gather(row_v, [jnp.zeros((16,), jnp.int32),
                                             codes_v[sl]])
                acc_s[0, sl] += v
                acc_q[sl] += v * v
                return cr
            lax.fori_loop(0, HB // 16, grp, 0)

    def fin(g, cr):
        sl = pl.ds(g * 16, 16)
        s = acc_s[0, sl]
        q = acc_q[sl]
        acc_s[0, sl] = 0.5 * (s * s - q)
        return cr
    lax.fori_loop(0, HB // 16, fin, 0)

    pltpu.sync_copy(acc_s, bi_out.at[pl.ds(e, 1), pl.ds(ch * HB, HB)])


_sc_pool = functools.partial(
    pl.kernel,
    out_type=jax.ShapeDtypeStruct((E, B), jnp.float32),
    mesh=plsc.VectorSubcoreMesh(core_axis_name="c", subcore_axis_name="s"),
    scratch_types=[
        pltpu.VMEM((1, V), jnp.float32),
        pltpu.VMEM((HB,), jnp.int32),
        pltpu.VMEM((1, HB), jnp.float32),
        pltpu.VMEM((HB,), jnp.float32),
        pltpu.SemaphoreType.DMA,
    ],
    compiler_params=pltpu.CompilerParams(needs_layout_passes=False),
)(_sc_body)


BM = 2048


def _mlp_body(dense_ref, bit_ref, w1a_ref, w1b_ref, b1_ref, w2_ref, b2_ref,
              w3_ref, b3_ref, out_ref):
    h = jnp.dot(dense_ref[...], w1a_ref[...], preferred_element_type=jnp.float32)
    h += lax.dot_general(bit_ref[...], w1b_ref[...],
                         (((0,), (0,)), ((), ())),
                         preferred_element_type=jnp.float32)
    h = jnp.maximum(h + b1_ref[...], 0.0)
    h = jnp.dot(h, w2_ref[...], preferred_element_type=jnp.float32)
    h = jnp.maximum(h + b2_ref[...], 0.0)
    out_ref[...] = (
        jnp.dot(h, w3_ref[...], preferred_element_type=jnp.float32)
        + b3_ref[...])


def _mlp(dense, bi_t, W1a, W1b, b1, W2, b2, W3, b3):
    grid = (B // BM,)
    full = lambda shape: pl.BlockSpec(shape, lambda i: (0, 0))
    return pl.pallas_call(
        _mlp_body,
        grid=grid,
        in_specs=[
            pl.BlockSpec((BM, ND), lambda i: (i, 0)),
            pl.BlockSpec((E, BM), lambda i: (0, i)),
            full((ND, H1)),
            full((E, H1)),
            full((1, H1)),
            full((H1, H2)),
            full((1, H2)),
            full((H2, OUT)),
            full((1, OUT)),
        ],
        out_specs=pl.BlockSpec((BM, OUT), lambda i: (i, 0)),
        out_shape=jax.ShapeDtypeStruct((B, OUT), jnp.float32),
    )(dense, bi_t, W1a, W1b, b1, W2, b2, W3, b3)


def kernel(target_x, tables, W1, b1, W2, b2, W3, b3):
    dense = target_x[:, :ND]
    sparse = target_x[:, ND:].astype(jnp.int32)
    codes_t = jnp.transpose(sparse, (1, 0))
    tt = jnp.transpose(tables, (0, 2, 1))

    bi_t = _sc_pool(tt, codes_t)

    return _mlp(dense, bi_t, W1[:ND], W1[ND:], b1[None, :], W2, b2[None, :],
                W3, b3[None, :])
